# baseline (speedup 1.0000x reference)
import functools

import jax
import jax.numpy as jnp
from jax.experimental import pallas as pl
from jax.experimental.pallas import tpu as pltpu


def _mlp_kernel(num_k, xt_ref, wt_ref, ct_ref, cwt_ref, pb_ref, cb_ref,
                jwt_ref, jb_ref, owt_ref, ob_ref, out_ref,
                acc_ref, ce_ref):
    k = pl.program_id(1)
    nk = pl.num_programs(1)
    k_blk = xt_ref.shape[0]

    @pl.when(k == 0)
    def _init():
        acc_ref[...] = jnp.zeros_like(acc_ref)
        ce_ref[...] = jax.nn.relu(
            jnp.dot(cwt_ref[...], ct_ref[...],
                    preferred_element_type=jnp.float32) + cb_ref[...])

    @pl.when(k < nk - 1)
    def _full_block():
        acc_ref[...] += jnp.dot(wt_ref[...].astype(jnp.bfloat16),
                                xt_ref[...].astype(jnp.bfloat16),
                                preferred_element_type=jnp.float32)

    @pl.when(k == nk - 1)
    def _epilogue():
        valid = num_k - (nk - 1) * k_blk
        xt = xt_ref[...]
        wt = wt_ref[...]
        xm = jax.lax.broadcasted_iota(jnp.int32, xt.shape, 0) < valid
        wm = jax.lax.broadcasted_iota(jnp.int32, wt.shape, 1) < valid
        acc_ref[...] += jnp.dot(
            jnp.where(wm, wt, 0.0).astype(jnp.bfloat16),
            jnp.where(xm, xt, 0.0).astype(jnp.bfloat16),
            preferred_element_type=jnp.float32)

        pet = jax.nn.relu(acc_ref[...] + pb_ref[...])
        jwt = jwt_ref[...]
        emb = pet.shape[0]
        joined_t = jax.nn.relu(
            jnp.dot(jwt[:, :emb], pet, preferred_element_type=jnp.float32)
            + jnp.dot(jwt[:, emb:], ce_ref[...],
                      preferred_element_type=jnp.float32)
            + jb_ref[...])
        out_ref[...] = (jnp.dot(owt_ref[...], joined_t,
                                preferred_element_type=jnp.float32)
                        + ob_ref[...])


@functools.partial(jax.jit, static_argnames=("k_blk",))
def _run(protein_input, compound_input, protein_W, protein_b, compound_W,
         compound_b, joined_W, joined_b, out_W, out_b, k_blk):
    batch, num_k = protein_input.shape
    emb = protein_W.shape[1]
    nfp = compound_input.shape[1]
    nk = -(-num_k // k_blk)

    nb = 2
    b_blk = batch // nb
    in_specs = [
        pl.BlockSpec((k_blk, b_blk), lambda b, k: (k, b)),
        pl.BlockSpec((emb, k_blk), lambda b, k: (0, k)),
        pl.BlockSpec((nfp, b_blk), lambda b, k: (0, b)),
        pl.BlockSpec((emb, nfp), lambda b, k: (0, 0)),
        pl.BlockSpec((emb, 1), lambda b, k: (0, 0)),
        pl.BlockSpec((emb, 1), lambda b, k: (0, 0)),
        pl.BlockSpec((emb, 2 * emb), lambda b, k: (0, 0)),
        pl.BlockSpec((emb, 1), lambda b, k: (0, 0)),
        pl.BlockSpec((1, emb), lambda b, k: (0, 0)),
        pl.BlockSpec((1, 1), lambda b, k: (0, 0)),
    ]

    out_t = pl.pallas_call(
        functools.partial(_mlp_kernel, num_k),
        grid=(nb, nk),
        in_specs=in_specs,
        out_specs=pl.BlockSpec((1, b_blk), lambda b, k: (0, b)),
        out_shape=jax.ShapeDtypeStruct((1, batch), jnp.float32),
        scratch_shapes=[
            pltpu.VMEM((emb, b_blk), jnp.float32),
            pltpu.VMEM((emb, b_blk), jnp.float32),
        ],
        compiler_params=pltpu.CompilerParams(
            dimension_semantics=("parallel", "arbitrary"),
        ),
    )(protein_input.T, protein_W.T, compound_input.T, compound_W.T,
      protein_b.reshape(emb, 1), compound_b.reshape(emb, 1),
      joined_W.T, joined_b.reshape(emb, 1), out_W.T, out_b.reshape(1, 1))
    return out_t.T


def kernel(protein_input, compound_input, protein_W, protein_b, compound_W,
           compound_b, joined_W, joined_b, out_W, out_b):
    return _run(protein_input, compound_input, protein_W, protein_b,
                compound_W, compound_b, joined_W, joined_b, out_W, out_b,
                k_blk=2048)


# speedup vs baseline: 1.1834x; 1.1834x over previous
import functools

import jax
import jax.numpy as jnp
from jax.experimental import pallas as pl
from jax.experimental.pallas import tpu as pltpu


def _mlp_kernel(num_k, xt_ref, wt_ref, ct_ref, cwt_ref, pb_ref, cb_ref,
                jwt_ref, jb_ref, owt_ref, ob_ref, out_ref,
                acc_ref, ce_ref):
    k = pl.program_id(0)
    nk = pl.num_programs(0)
    k_blk = xt_ref.shape[0]

    @pl.when(k == 0)
    def _init():
        acc_ref[...] = jnp.zeros_like(acc_ref)
        ce_ref[...] = jax.nn.relu(
            jnp.dot(cwt_ref[...], ct_ref[...],
                    preferred_element_type=jnp.float32) + cb_ref[...])

    @pl.when(k < nk - 1)
    def _full_block():
        acc_ref[...] += jnp.dot(wt_ref[...].astype(jnp.bfloat16),
                                xt_ref[...].astype(jnp.bfloat16),
                                preferred_element_type=jnp.float32)

    @pl.when(k == nk - 1)
    def _epilogue():
        valid = num_k - (nk - 1) * k_blk
        xt = xt_ref[...]
        wt = wt_ref[...]
        xm = jax.lax.broadcasted_iota(jnp.int32, xt.shape, 0) < valid
        wm = jax.lax.broadcasted_iota(jnp.int32, wt.shape, 1) < valid
        acc_ref[...] += jnp.dot(
            jnp.where(wm, wt, 0.0).astype(jnp.bfloat16),
            jnp.where(xm, xt, 0.0).astype(jnp.bfloat16),
            preferred_element_type=jnp.float32)

        pet = jax.nn.relu(acc_ref[...] + pb_ref[...])
        jwt = jwt_ref[...]
        emb = pet.shape[0]
        joined_t = jax.nn.relu(
            jnp.dot(jwt[:, :emb], pet, preferred_element_type=jnp.float32)
            + jnp.dot(jwt[:, emb:], ce_ref[...],
                      preferred_element_type=jnp.float32)
            + jb_ref[...])
        out_ref[...] = (jnp.dot(owt_ref[...], joined_t,
                                preferred_element_type=jnp.float32)
                        + ob_ref[...])


@functools.partial(jax.jit, static_argnames=("k_blk",))
def _run(protein_input, compound_input, protein_W, protein_b, compound_W,
         compound_b, joined_W, joined_b, out_W, out_b, k_blk):
    batch, num_k = protein_input.shape
    emb = protein_W.shape[1]
    nfp = compound_input.shape[1]
    nk = -(-num_k // k_blk)

    in_specs = [
        pl.BlockSpec((k_blk, batch), lambda k: (k, 0)),
        pl.BlockSpec((emb, k_blk), lambda k: (0, k)),
        pl.BlockSpec((nfp, batch), lambda k: (0, 0)),
        pl.BlockSpec((emb, nfp), lambda k: (0, 0)),
        pl.BlockSpec((emb, 1), lambda k: (0, 0)),
        pl.BlockSpec((emb, 1), lambda k: (0, 0)),
        pl.BlockSpec((emb, 2 * emb), lambda k: (0, 0)),
        pl.BlockSpec((emb, 1), lambda k: (0, 0)),
        pl.BlockSpec((1, emb), lambda k: (0, 0)),
        pl.BlockSpec((1, 1), lambda k: (0, 0)),
    ]

    out_t = pl.pallas_call(
        functools.partial(_mlp_kernel, num_k),
        grid=(nk,),
        in_specs=in_specs,
        out_specs=pl.BlockSpec((1, batch), lambda k: (0, 0)),
        out_shape=jax.ShapeDtypeStruct((1, batch), jnp.float32),
        scratch_shapes=[
            pltpu.VMEM((emb, batch), jnp.float32),
            pltpu.VMEM((emb, batch), jnp.float32),
        ],
        compiler_params=pltpu.CompilerParams(
            dimension_semantics=("arbitrary",),
        ),
    )(protein_input.T, protein_W.T, compound_input.T, compound_W.T,
      protein_b.reshape(emb, 1), compound_b.reshape(emb, 1),
      joined_W.T, joined_b.reshape(emb, 1), out_W.T, out_b.reshape(1, 1))
    return out_t.T


def kernel(protein_input, compound_input, protein_W, protein_b, compound_W,
           compound_b, joined_W, joined_b, out_W, out_b):
    return _run(protein_input, compound_input, protein_W, protein_b,
                compound_W, compound_b, joined_W, joined_b, out_W, out_b,
                k_blk=2048)
